# baseline (speedup 1.0000x reference)
import functools

import jax
import jax.numpy as jnp
from jax import lax
from jax.experimental import pallas as pl
from jax.experimental.pallas import tpu as pltpu
from jax.experimental.pallas import tpu_sc as plsc

_CHUNK = 128
_GROUP = 5


@functools.lru_cache(maxsize=None)
def _build_gather(n_rows, d):
    info = plsc.get_sparse_core_info()
    nc, ns = info.num_cores, info.num_subcores
    nw = nc * ns
    per_w = n_rows // nw
    n_chunks = per_w // _CHUNK
    n_super = n_chunks // _GROUP
    assert per_w * nw == n_rows
    assert n_chunks * _CHUNK == per_w
    assert n_super * _GROUP == n_chunks

    mesh = plsc.VectorSubcoreMesh(core_axis_name="c", subcore_axis_name="s")

    scratch = (
        [pltpu.VMEM((n_chunks, _CHUNK), jnp.int32)]
        + [pltpu.VMEM((_CHUNK, d), jnp.float32) for _ in range(_GROUP)]
        + [pltpu.SemaphoreType.DMA for _ in range(2 * _GROUP)]
    )

    @functools.partial(
        pl.kernel,
        mesh=mesh,
        out_type=jax.ShapeDtypeStruct((n_rows, d), jnp.float32),
        scratch_types=scratch,
    )
    def gather(idx_hbm, table_hbm, out_hbm, idx_v, *rest):
        bufs = rest[:_GROUP]
        gsems = rest[_GROUP:2 * _GROUP]
        ssems = rest[2 * _GROUP:]

        wid = lax.axis_index("s") * nc + lax.axis_index("c")
        cbase = wid * n_chunks
        pltpu.sync_copy(idx_hbm.at[wid], idx_v)

        def body(s, carry):
            j0 = s * _GROUP
            hg = [
                pltpu.async_copy(table_hbm.at[idx_v.at[j0 + b]], bufs[b],
                                 gsems[b])
                for b in range(_GROUP)
            ]
            hs = []
            for b in range(_GROUP):
                hg[b].wait()
                hs.append(pltpu.async_copy(
                    bufs[b],
                    out_hbm.at[pl.ds((cbase + j0 + b) * _CHUNK, _CHUNK)],
                    ssems[b]))
            for h in hs:
                h.wait()
            return carry

        lax.fori_loop(0, n_super, body, 0)

    return gather


_RB = 8


def _relayout_body(flat_ref, out_ref):
    for r in range(_RB):
        out_ref[r] = flat_ref[pl.ds(r * out_ref.shape[1], out_ref.shape[1])]


@functools.lru_cache(maxsize=None)
def _build_relayout(bsz, seq, d):
    return pl.pallas_call(
        _relayout_body,
        grid=(bsz // _RB,),
        in_specs=[pl.BlockSpec((_RB * seq, d), lambda i: (i, 0))],
        out_specs=pl.BlockSpec((_RB, seq, d), lambda i: (i, 0, 0)),
        out_shape=jax.ShapeDtypeStruct((bsz, seq, d), jnp.float32),
    )


def kernel(input, table):
    bsz, seq = input.shape
    _, d = table.shape
    n = bsz * seq
    info = plsc.get_sparse_core_info()
    nw = info.num_cores * info.num_subcores
    idx = input.reshape(nw, n // (nw * _CHUNK), _CHUNK).astype(jnp.int32)
    flat = _build_gather(n, d)(idx, table)
    return _build_relayout(bsz, seq, d)(flat)


# speedup vs baseline: 2.7926x; 2.7926x over previous
import functools

import jax
import jax.numpy as jnp
from jax import lax
from jax.experimental import pallas as pl
from jax.experimental.pallas import tpu as pltpu
from jax.experimental.pallas import tpu_sc as plsc

_NB = 4
_NBUF = 4
_SEQ_PAD = 56


@functools.lru_cache(maxsize=None)
def _build_gather(bsz, seq, d):
    info = plsc.get_sparse_core_info()
    nc, ns = info.num_cores, info.num_subcores
    nw = nc * ns
    b_per_w = bsz // nw
    n_chunks = b_per_w // _NB
    n_super = n_chunks // _NBUF
    assert b_per_w * nw == bsz
    assert n_chunks * _NB == b_per_w
    assert n_super * _NBUF == n_chunks
    assert seq <= _SEQ_PAD and _SEQ_PAD % 8 == 0

    mesh = plsc.VectorSubcoreMesh(core_axis_name="c", subcore_axis_name="s")

    scratch = (
        [pltpu.VMEM((b_per_w, _SEQ_PAD), jnp.int32)]
        + [pltpu.VMEM((_NB, seq, d), jnp.float32) for _ in range(_NBUF)]
        + [pltpu.SemaphoreType.DMA for _ in range(2 * _NBUF)]
    )

    @functools.partial(
        pl.kernel,
        mesh=mesh,
        out_type=jax.ShapeDtypeStruct((bsz, seq, d), jnp.float32),
        scratch_types=scratch,
    )
    def gather(idx_hbm, table_hbm, out_hbm, idx_v, *rest):
        bufs = rest[:_NBUF]
        gsems = rest[_NBUF:2 * _NBUF]
        ssems = rest[2 * _NBUF:]

        wid = lax.axis_index("s") * nc + lax.axis_index("c")
        base = wid * b_per_w
        pltpu.sync_copy(idx_hbm.at[pl.ds(base, b_per_w)], idx_v)

        def body(s, carry):
            c0 = s * _NBUF
            hg = []
            for k in range(_NBUF):
                for r in range(_NB):
                    row = (c0 + k) * _NB + r
                    hg.append(pltpu.async_copy(
                        table_hbm.at[idx_v.at[row, pl.ds(0, seq)]],
                        bufs[k].at[r], gsems[k]))
            hs = []
            for k in range(_NBUF):
                for r in range(_NB):
                    hg[k * _NB + r].wait()
                hs.append(pltpu.async_copy(
                    bufs[k], out_hbm.at[pl.ds(base + (c0 + k) * _NB, _NB)],
                    ssems[k]))
            for h in hs:
                h.wait()
            return carry

        lax.fori_loop(0, n_super, body, 0)

    return gather


def kernel(input, table):
    bsz, seq = input.shape
    _, d = table.shape
    idx = jnp.pad(input.astype(jnp.int32), ((0, 0), (0, _SEQ_PAD - seq)))
    return _build_gather(bsz, seq, d)(idx, table)
